# baseline (speedup 1.0000x reference)
import functools

import jax
import jax.numpy as jnp
from jax import lax
from jax.experimental import pallas as pl
from jax.experimental.pallas import tpu as pltpu
from jax.experimental.pallas import tpu_sc as plsc

CONF_THRES = 0.25
IOU_THRES = 0.45
MAX_DET = 300

N = 20000
L = 16
NT = 16
PER = 1280
NPAD = NT * PER
CHUNKS = PER // L
OUT_ROWS = 304
BIG = 2**30


def _iota16():
    return lax.broadcasted_iota(jnp.int32, (L,), 0)


def _spf(x):
    return jnp.full((L,), x, jnp.float32)


def _spi(x):
    return jnp.full((L,), x, jnp.int32)


def _sc_body(x1h, y1h, x2h, y2h, sh, outh,
             x1v, y1v, x2v, y2v, a2v, wv, pubv, gv, obuf, pub_sp):
    cid = lax.axis_index("c")
    sid = lax.axis_index("s")

    @pl.when(cid == 0)
    def _run():
        base = sid * PER
        pltpu.sync_copy(x1h.at[pl.ds(base, PER)], x1v)
        pltpu.sync_copy(y1h.at[pl.ds(base, PER)], y1v)
        pltpu.sync_copy(x2h.at[pl.ds(base, PER)], x2v)
        pltpu.sync_copy(y2h.at[pl.ds(base, PER)], y2v)
        pltpu.sync_copy(sh.at[pl.ds(base, PER)], wv)
        iota = _iota16()
        run0 = jnp.full((L,), -jnp.inf, jnp.float32)

        def init_chunk(k, carry):
            run, idx = carry
            sl = pl.ds(k * L, L)
            x1c = x1v[sl]
            y1c = y1v[sl]
            x2c = x2v[sl]
            y2c = y2v[sl]
            a2v[sl] = (x2c - x1c) * (y2c - y1c)
            s = wv[sl]
            w = jnp.where(s >= CONF_THRES, s, -jnp.inf)
            wv[sl] = w
            take = w > run
            run = jnp.where(take, w, run)
            idx = jnp.where(take, k * L + iota, idx)
            return run, idx

        run, idx = lax.fori_loop(0, CHUNKS, init_chunk, (run0, iota))
        lv = jnp.max(run)
        li = jnp.min(jnp.where(run == lv, idx, BIG))

        def round_body(i, carry):
            lv, li = carry
            lidx = _spi(li)
            pubv[pl.ds(0, L)] = _spf(lv)
            pubv[pl.ds(L, L)] = plsc.bitcast(_spi(li + base), jnp.float32)
            pubv[pl.ds(2 * L, L)] = plsc.load_gather(x1v, [lidx])
            pubv[pl.ds(3 * L, L)] = plsc.load_gather(y1v, [lidx])
            pubv[pl.ds(4 * L, L)] = plsc.load_gather(x2v, [lidx])
            pubv[pl.ds(5 * L, L)] = plsc.load_gather(y2v, [lidx])
            pltpu.sync_copy(pubv, pub_sp.at[sid])
            plsc.subcore_barrier()
            pltpu.sync_copy(pub_sp, gv)
            plsc.subcore_barrier()

            rows = iota
            mall = plsc.load_gather(gv, [rows, _spi(0)])
            iall = plsc.bitcast(plsc.load_gather(gv, [rows, _spi(L)]),
                                jnp.int32)
            gmax = jnp.max(mall)
            gj = jnp.min(jnp.where(mall == gmax, iall, BIG))
            sel = iall == gj
            x1g = jnp.sum(jnp.where(
                sel, plsc.load_gather(gv, [rows, _spi(2 * L)]), 0.0))
            y1g = jnp.sum(jnp.where(
                sel, plsc.load_gather(gv, [rows, _spi(3 * L)]), 0.0))
            x2g = jnp.sum(jnp.where(
                sel, plsc.load_gather(gv, [rows, _spi(4 * L)]), 0.0))
            y2g = jnp.sum(jnp.where(
                sel, plsc.load_gather(gv, [rows, _spi(5 * L)]), 0.0))

            @pl.when(sid == 0)
            def _write():
                finite = gmax > -jnp.inf
                x1o = jnp.where(finite, x1g, 0.0)
                y1o = jnp.where(finite, y1g, 0.0)
                x2o = jnp.where(finite, x2g, 0.0)
                y2o = jnp.where(finite, y2g, 0.0)
                sco = jnp.where(finite, gmax, 0.0)
                row = jnp.where(iota == 0, x1o,
                      jnp.where(iota == 1, y1o,
                      jnp.where(iota == 2, x2o,
                      jnp.where(iota == 3, y2o,
                      jnp.where(iota == 4, sco, 0.0)))))
                obuf[pl.ds(i * L, L)] = row

            area1 = (x2g - x1g) * (y2g - y1g)

            def supp_chunk(k, carry2):
                run, idx = carry2
                sl = pl.ds(k * L, L)
                x1c = x1v[sl]
                y1c = y1v[sl]
                x2c = x2v[sl]
                y2c = y2v[sl]
                w = wv[sl]
                ltx = jnp.maximum(x1g, x1c)
                lty = jnp.maximum(y1g, y1c)
                rbx = jnp.minimum(x2g, x2c)
                rby = jnp.minimum(y2g, y2c)
                iw = jnp.maximum(rbx - ltx, 0.0)
                ih = jnp.maximum(rby - lty, 0.0)
                inter = iw * ih
                iou = inter / (area1 + a2v[sl] - inter + 1e-9)
                glin = base + k * L + iota
                w2 = jnp.where((iou > IOU_THRES) | (glin == gj),
                               -jnp.inf, w)
                wv[sl] = w2
                take = w2 > run
                run = jnp.where(take, w2, run)
                idx = jnp.where(take, k * L + iota, idx)
                return run, idx

            run, idx = lax.fori_loop(0, CHUNKS, supp_chunk, (run0, iota))
            lv2 = jnp.max(run)
            li2 = jnp.min(jnp.where(run == lv2, idx, BIG))
            return lv2, li2

        lax.fori_loop(0, MAX_DET, round_body, (lv, li))

        @pl.when(sid == 0)
        def _finish():
            pltpu.sync_copy(obuf, outh)


@jax.jit
def kernel(boxes, scores):
    bp = jnp.pad(boxes, ((0, NPAD - N), (0, 0)))
    sp = jnp.pad(scores, (0, NPAD - N))
    mesh = plsc.VectorSubcoreMesh(core_axis_name="c", subcore_axis_name="s",
                                  num_cores=2, num_subcores=NT)
    call = pl.kernel(
        _sc_body,
        out_type=jax.ShapeDtypeStruct((OUT_ROWS * L,), jnp.float32),
        mesh=mesh,
        compiler_params=pltpu.CompilerParams(needs_layout_passes=False),
        scratch_types=[
            pltpu.VMEM((PER,), jnp.float32),
            pltpu.VMEM((PER,), jnp.float32),
            pltpu.VMEM((PER,), jnp.float32),
            pltpu.VMEM((PER,), jnp.float32),
            pltpu.VMEM((PER,), jnp.float32),
            pltpu.VMEM((PER,), jnp.float32),
            pltpu.VMEM((8 * L,), jnp.float32),
            pltpu.VMEM((NT, 8 * L), jnp.float32),
            pltpu.VMEM((OUT_ROWS * L,), jnp.float32),
            pltpu.VMEM_SHARED((NT, 8 * L), jnp.float32),
        ],
    )
    out = call(bp[:, 0], bp[:, 1], bp[:, 2], bp[:, 3], sp)
    return out.reshape(OUT_ROWS, L)[:MAX_DET, :5]


# speedup vs baseline: 2.2962x; 2.2962x over previous
import functools

import jax
import jax.numpy as jnp
from jax import lax
from jax.experimental import pallas as pl
from jax.experimental.pallas import tpu as pltpu
from jax.experimental.pallas import tpu_sc as plsc

CONF_THRES = 0.25
IOU_THRES = 0.45
MAX_DET = 300

N = 20000
L = 16
NT = 16
PER = 1280
NPAD = NT * PER
CHUNKS = PER // L
OUT_ROWS = 304
BIG = 2**30


def _iota16():
    return lax.broadcasted_iota(jnp.int32, (L,), 0)


def _spf(x):
    return jnp.full((L,), x, jnp.float32)


def _spi(x):
    return jnp.full((L,), x, jnp.int32)


def _sc_body(x1h, y1h, x2h, y2h, sh, outh,
             x1v, y1v, x2v, y2v, a2v, wv, pubv, gv, obuf, pub_sp):
    cid = lax.axis_index("c")
    sid = lax.axis_index("s")

    @pl.when(cid == 0)
    def _run():
        base = sid * PER
        pltpu.sync_copy(x1h.at[pl.ds(base, PER)], x1v)
        pltpu.sync_copy(y1h.at[pl.ds(base, PER)], y1v)
        pltpu.sync_copy(x2h.at[pl.ds(base, PER)], x2v)
        pltpu.sync_copy(y2h.at[pl.ds(base, PER)], y2v)
        pltpu.sync_copy(sh.at[pl.ds(base, PER)], wv)
        iota = _iota16()
        run0 = jnp.full((L,), -jnp.inf, jnp.float32)

        @plsc.parallel_loop(0, CHUNKS, unroll=4, carry=(run0, iota))
        def init_loop(k, carry):
            run, idx = carry
            sl = pl.ds(k * L, L)
            x1c = x1v[sl]
            y1c = y1v[sl]
            x2c = x2v[sl]
            y2c = y2v[sl]
            a2v[sl] = (x2c - x1c) * (y2c - y1c)
            s = wv[sl]
            w = jnp.where(s >= CONF_THRES, s, -jnp.inf)
            wv[sl] = w
            ci = k * L + iota
            take = (w > run) | ((w == run) & (ci < idx))
            run = jnp.where(take, w, run)
            idx = jnp.where(take, ci, idx)
            return run, idx

        run, idx = init_loop
        lv = jnp.max(run)
        li = jnp.min(jnp.where(run == lv, idx, BIG))

        def round_body(i, carry):
            lv, li = carry
            lidx = _spi(li)
            pubv[pl.ds(0, L)] = _spf(lv)
            pubv[pl.ds(L, L)] = plsc.bitcast(_spi(li + base), jnp.float32)
            pubv[pl.ds(2 * L, L)] = plsc.load_gather(x1v, [lidx])
            pubv[pl.ds(3 * L, L)] = plsc.load_gather(y1v, [lidx])
            pubv[pl.ds(4 * L, L)] = plsc.load_gather(x2v, [lidx])
            pubv[pl.ds(5 * L, L)] = plsc.load_gather(y2v, [lidx])
            slot = i & 1
            pltpu.sync_copy(pubv, pub_sp.at[slot, sid])
            plsc.subcore_barrier()
            pltpu.sync_copy(pub_sp.at[slot], gv)

            rows = iota
            mall = plsc.load_gather(gv, [rows, _spi(0)])
            iall = plsc.bitcast(plsc.load_gather(gv, [rows, _spi(L)]),
                                jnp.int32)
            gmax = jnp.max(mall)
            gj = jnp.min(jnp.where(mall == gmax, iall, BIG))
            sel = iall == gj
            x1g = jnp.sum(jnp.where(
                sel, plsc.load_gather(gv, [rows, _spi(2 * L)]), 0.0))
            y1g = jnp.sum(jnp.where(
                sel, plsc.load_gather(gv, [rows, _spi(3 * L)]), 0.0))
            x2g = jnp.sum(jnp.where(
                sel, plsc.load_gather(gv, [rows, _spi(4 * L)]), 0.0))
            y2g = jnp.sum(jnp.where(
                sel, plsc.load_gather(gv, [rows, _spi(5 * L)]), 0.0))

            @pl.when(sid == 0)
            def _write():
                finite = gmax > -jnp.inf
                x1o = jnp.where(finite, x1g, 0.0)
                y1o = jnp.where(finite, y1g, 0.0)
                x2o = jnp.where(finite, x2g, 0.0)
                y2o = jnp.where(finite, y2g, 0.0)
                sco = jnp.where(finite, gmax, 0.0)
                row = jnp.where(iota == 0, x1o,
                      jnp.where(iota == 1, y1o,
                      jnp.where(iota == 2, x2o,
                      jnp.where(iota == 3, y2o,
                      jnp.where(iota == 4, sco, 0.0)))))
                obuf[pl.ds(i * L, L)] = row

            area1 = (x2g - x1g) * (y2g - y1g)

            @plsc.parallel_loop(0, CHUNKS, unroll=4, carry=(run0, iota))
            def supp_loop(k, carry2):
                run, idx = carry2
                sl = pl.ds(k * L, L)
                x1c = x1v[sl]
                y1c = y1v[sl]
                x2c = x2v[sl]
                y2c = y2v[sl]
                w = wv[sl]
                ltx = jnp.maximum(x1g, x1c)
                lty = jnp.maximum(y1g, y1c)
                rbx = jnp.minimum(x2g, x2c)
                rby = jnp.minimum(y2g, y2c)
                iw = jnp.maximum(rbx - ltx, 0.0)
                ih = jnp.maximum(rby - lty, 0.0)
                inter = iw * ih
                iou = inter / (area1 + a2v[sl] - inter + 1e-9)
                ci = k * L + iota
                w2 = jnp.where((iou > IOU_THRES) | ((base + ci) == gj),
                               -jnp.inf, w)
                wv[sl] = w2
                take = (w2 > run) | ((w2 == run) & (ci < idx))
                run = jnp.where(take, w2, run)
                idx = jnp.where(take, ci, idx)
                return run, idx

            run, idx = supp_loop
            lv2 = jnp.max(run)
            li2 = jnp.min(jnp.where(run == lv2, idx, BIG))
            return lv2, li2

        lax.fori_loop(0, MAX_DET, round_body, (lv, li))

        @pl.when(sid == 0)
        def _finish():
            pltpu.sync_copy(obuf, outh)


@jax.jit
def kernel(boxes, scores):
    bp = jnp.pad(boxes, ((0, NPAD - N), (0, 0)))
    sp = jnp.pad(scores, (0, NPAD - N))
    mesh = plsc.VectorSubcoreMesh(core_axis_name="c", subcore_axis_name="s",
                                  num_cores=2, num_subcores=NT)
    call = pl.kernel(
        _sc_body,
        out_type=jax.ShapeDtypeStruct((OUT_ROWS * L,), jnp.float32),
        mesh=mesh,
        compiler_params=pltpu.CompilerParams(needs_layout_passes=False),
        scratch_types=[
            pltpu.VMEM((PER,), jnp.float32),
            pltpu.VMEM((PER,), jnp.float32),
            pltpu.VMEM((PER,), jnp.float32),
            pltpu.VMEM((PER,), jnp.float32),
            pltpu.VMEM((PER,), jnp.float32),
            pltpu.VMEM((PER,), jnp.float32),
            pltpu.VMEM((8 * L,), jnp.float32),
            pltpu.VMEM((NT, 8 * L), jnp.float32),
            pltpu.VMEM((OUT_ROWS * L,), jnp.float32),
            pltpu.VMEM_SHARED((2, NT, 8 * L), jnp.float32),
        ],
    )
    out = call(bp[:, 0], bp[:, 1], bp[:, 2], bp[:, 3], sp)
    return out.reshape(OUT_ROWS, L)[:MAX_DET, :5]
